# baseline (speedup 1.0000x reference)
import functools

import jax
import jax.numpy as jnp
from jax import lax
from jax.experimental import pallas as pl
from jax.experimental.pallas import tpu as pltpu
from jax.experimental.pallas import tpu_sc as plsc

_P = 2147483647
_M = 100000
_A = (48271, 16807)
_B = (12345, 67890)
_D = 64
_NC = 2
_NS = 16
_NW = _NC * _NS
_L = 16
_V = 1000000


def _hashes(xv):
    i32 = jnp.int32
    xl = xv & i32(0x7FFF)
    xh = lax.shift_right_logical(xv, i32(15))
    out = []
    for a, b in zip(_A, _B):
        u = xh * i32(a)
        t = (lax.shift_right_logical(u, i32(16))
             + lax.shift_left(u & i32(0xFFFF), i32(15))
             + xl * i32(a) + i32(b))
        r = (t & i32(0x7FFFFFFF)) + lax.shift_right_logical(t, i32(31))
        r = jnp.where(r == i32(_P), i32(0), jnp.where(r < i32(0), i32(1), r))
        h = lax.rem(r, i32(_M)) + i32(4)
        out.append(jnp.where(xv < i32(4), xv, h))
    return out


def _make_lookup(seq, batch):
    bpw = batch // _NW
    assert bpw * _NW == batch and bpw % _L == 0 and bpw <= 128
    assert seq % 2 == 0

    mesh = plsc.VectorSubcoreMesh(
        core_axis_name="c", subcore_axis_name="s",
        num_cores=_NC, num_subcores=_NS)

    @functools.partial(
        pl.kernel,
        out_type=jax.ShapeDtypeStruct(
            (seq, _D // 8, batch // 128, 8, 128), jnp.float32),
        mesh=mesh,
        scratch_types=[
            pltpu.VMEM((seq, bpw), jnp.int32),
            pltpu.VMEM((seq, bpw), jnp.int32),
            pltpu.VMEM((seq, bpw), jnp.int32),
            pltpu.VMEM((seq, bpw), jnp.int32),
            pltpu.VMEM((2, bpw, _L), jnp.float32),
            pltpu.VMEM((2, bpw, _D), jnp.float32),
            pltpu.VMEM((2, bpw, _D), jnp.float32),
            pltpu.VMEM((2, _D // 8, 8, bpw), jnp.float32),
            pltpu.SemaphoreType.DMA,
            pltpu.SemaphoreType.DMA,
            pltpu.SemaphoreType.DMA,
            pltpu.SemaphoreType.DMA,
            pltpu.SemaphoreType.DMA,
            pltpu.SemaphoreType.DMA,
            pltpu.SemaphoreType.DMA,
            pltpu.SemaphoreType.DMA,
        ],
        compiler_params=pltpu.CompilerParams(
            needs_layout_passes=False, use_tc_tiling_on_sc=False),
    )
    def lookup(xt_hbm, emb_hbm, wi_hbm, out_hbm,
               idx_s, h0_s, h1_s, g_s, wraw, r0_v, r1_v, o_v,
               sw0, sw1, s00, s01, s10, s11, so0, so1):
        i32 = jnp.int32
        wid = lax.axis_index("s") * i32(_NC) + lax.axis_index("c")
        b0 = pl.multiple_of(wid * i32(bpw), bpw)
        sems = ((sw0, s00, s10, so0), (sw1, s01, s11, so1))

        pltpu.sync_copy(xt_hbm.at[:, pl.ds(b0, bpw)], idx_s)

        def hash_body(_, r):
            for k in range(bpw // _L):
                sl = pl.ds(k * _L, _L)
                xv = idx_s[r, sl]
                h0, h1 = _hashes(xv)
                h0_s[r, sl] = h0
                h1_s[r, sl] = h1
                g_s[r, sl] = lax.shift_right_logical(xv, i32(3))
            return r + i32(1)

        lax.fori_loop(0, seq, hash_body, i32(0))

        def gather_args(r, p):
            return ((wi_hbm.at[g_s.at[r]], wraw.at[i32(p)], sems[p][0]),
                    (emb_hbm.at[h0_s.at[r]], r0_v.at[i32(p)], sems[p][1]),
                    (emb_hbm.at[h1_s.at[r]], r1_v.at[i32(p)], sems[p][2]))

        def issue_gathers(r, p):
            for args in gather_args(r, p):
                pltpu.async_copy(*args)

        def wait_gathers(r, p):
            for args in gather_args(r, p):
                pltpu.make_async_copy(*args).wait()

        def out_args(r, p):
            return (o_v.at[i32(p)], out_hbm.at[r, :, wid, :, :],
                    sems[p][3])

        def combine(r, p):
            iota = lax.iota(i32, _L)

            def grp_body(g2, koff):
                koff = pl.multiple_of(koff, _L)
                tokv = iota + koff
                ksl = pl.ds(koff, _L)
                col = idx_s[r, ksl] & i32(7)
                w0 = plsc.load_gather(wraw.at[i32(p)], [tokv, col])
                w1 = plsc.load_gather(wraw.at[i32(p)], [tokv, col + i32(8)])
                for d in range(_D):
                    dspl = jnp.full((_L,), d, i32)
                    g0 = plsc.load_gather(r0_v.at[i32(p)], [tokv, dspl])
                    g1 = plsc.load_gather(r1_v.at[i32(p)], [tokv, dspl])
                    o_v[i32(p), d // 8, d % 8, ksl] = w0 * g0 + w1 * g1
                return koff + i32(_L)

            lax.fori_loop(0, bpw // _L, grp_body, i32(0))

        issue_gathers(i32(0), 0)

        def pair_body(q, r):
            issue_gathers(r + i32(1), 1)
            wait_gathers(r, 0)

            @pl.when(r > i32(0))
            def _():
                pltpu.make_async_copy(*out_args(r - i32(2), 0)).wait()
            combine(r, 0)
            pltpu.async_copy(*out_args(r, 0))
            @pl.when(r + i32(2) < i32(seq))
            def _():
                issue_gathers(r + i32(2), 0)
            wait_gathers(r + i32(1), 1)

            @pl.when(r > i32(0))
            def _():
                pltpu.make_async_copy(*out_args(r - i32(1), 1)).wait()
            combine(r + i32(1), 1)
            pltpu.async_copy(*out_args(r + i32(1), 1))
            return r + i32(2)

        lax.fori_loop(0, seq // 2, pair_body, i32(0))
        pltpu.make_async_copy(*out_args(i32(seq - 2), 0)).wait()
        pltpu.make_async_copy(*out_args(i32(seq - 1), 1)).wait()

    return lookup


def kernel(x, shared_embedding, hash_weights):
    b, s = x.shape
    xt = x.T.astype(jnp.int32)
    wi = hash_weights.reshape(-1, 8, 2).transpose(0, 2, 1).reshape(-1, _L)
    lookup = _make_lookup(s, b)
    out5 = lookup(xt, shared_embedding, wi)
    return jnp.transpose(out5, (2, 4, 0, 1, 3)).reshape(b, s, _D)


# speedup vs baseline: 1.8069x; 1.8069x over previous
import functools

import jax
import jax.numpy as jnp
from jax import lax
from jax.experimental import pallas as pl
from jax.experimental.pallas import tpu as pltpu
from jax.experimental.pallas import tpu_sc as plsc

_P = 2147483647
_M = 100000
_A = (48271, 16807)
_B = (12345, 67890)
_D = 64
_NC = 2
_NS = 16
_NW = _NC * _NS
_L = 16
_V = 1000000


def _hashes(xv):
    i32 = jnp.int32
    xl = xv & i32(0x7FFF)
    xh = lax.shift_right_logical(xv, i32(15))
    out = []
    for a, b in zip(_A, _B):
        u = xh * i32(a)
        t = (lax.shift_right_logical(u, i32(16))
             + lax.shift_left(u & i32(0xFFFF), i32(15))
             + xl * i32(a) + i32(b))
        r = (t & i32(0x7FFFFFFF)) + lax.shift_right_logical(t, i32(31))
        r = jnp.where(r == i32(_P), i32(0), jnp.where(r < i32(0), i32(1), r))
        h = lax.rem(r, i32(_M)) + i32(4)
        out.append(jnp.where(xv < i32(4), xv, h))
    return out


def _make_lookup(seq, batch):
    bpw = batch // _NW
    assert bpw * _NW == batch and bpw % _L == 0 and bpw <= 128
    assert seq % 2 == 0

    mesh = plsc.VectorSubcoreMesh(
        core_axis_name="c", subcore_axis_name="s",
        num_cores=_NC, num_subcores=_NS)

    @functools.partial(
        pl.kernel,
        out_type=jax.ShapeDtypeStruct(
            (seq, _D // 8, batch // 128, 8, 128), jnp.float32),
        mesh=mesh,
        scratch_types=[
            pltpu.VMEM((seq, bpw), jnp.int32),
            pltpu.VMEM((seq, bpw), jnp.int32),
            pltpu.VMEM((seq, bpw), jnp.int32),
            pltpu.VMEM((seq, bpw), jnp.int32),
            pltpu.VMEM((2, bpw, _L), jnp.float32),
            pltpu.VMEM((2, bpw, _D), jnp.float32),
            pltpu.VMEM((2, bpw, _D), jnp.float32),
            pltpu.VMEM((2, _D // 8, 8, bpw), jnp.float32),
            pltpu.SemaphoreType.DMA,
            pltpu.SemaphoreType.DMA,
            pltpu.SemaphoreType.DMA,
            pltpu.SemaphoreType.DMA,
            pltpu.SemaphoreType.DMA,
            pltpu.SemaphoreType.DMA,
            pltpu.SemaphoreType.DMA,
            pltpu.SemaphoreType.DMA,
        ],
        compiler_params=pltpu.CompilerParams(
            needs_layout_passes=False, use_tc_tiling_on_sc=False),
    )
    def lookup(xt_hbm, emb_hbm, wi_hbm, out_hbm,
               idx_s, h0_s, h1_s, g_s, wraw, r0_v, r1_v, o_v,
               sw0, sw1, s00, s01, s10, s11, so0, so1):
        i32 = jnp.int32
        wid = lax.axis_index("s") * i32(_NC) + lax.axis_index("c")
        b0 = pl.multiple_of(wid * i32(bpw), bpw)
        sems = ((sw0, s00, s10, so0), (sw1, s01, s11, so1))

        pltpu.sync_copy(xt_hbm.at[:, pl.ds(b0, bpw)], idx_s)

        def hash_body(_, r):
            for k in range(bpw // _L):
                sl = pl.ds(k * _L, _L)
                xv = idx_s[r, sl]
                h0, h1 = _hashes(xv)
                h0_s[r, sl] = h0
                h1_s[r, sl] = h1
                g_s[r, sl] = lax.shift_right_logical(xv, i32(3))
            return r + i32(1)

        lax.fori_loop(0, seq, hash_body, i32(0))

        def gather_args(r, p):
            return ((wi_hbm.at[g_s.at[r]], wraw.at[i32(p)], sems[p][0]),
                    (emb_hbm.at[h0_s.at[r]], r0_v.at[i32(p)], sems[p][1]),
                    (emb_hbm.at[h1_s.at[r]], r1_v.at[i32(p)], sems[p][2]))

        def issue_gathers(r, p):
            for args in gather_args(r, p):
                pltpu.async_copy(*args)

        def wait_gathers(r, p):
            for args in gather_args(r, p):
                pltpu.make_async_copy(*args).wait()

        def out_args(r, p):
            return (o_v.at[i32(p)], out_hbm.at[r, :, wid, :, :],
                    sems[p][3])

        def combine(r, p):
            iota = lax.iota(i32, _L)
            diags = [(iota + i32(j)) & i32(_L - 1) for j in range(_L)]

            def grp_body(g2, koff):
                koff = pl.multiple_of(koff, _L)
                tokv = iota + koff
                ksl = pl.ds(koff, _L)
                col = idx_s[r, ksl] & i32(7)
                w0 = plsc.load_gather(wraw.at[i32(p)], [tokv, col])
                w1 = plsc.load_gather(wraw.at[i32(p)], [tokv, col + i32(8)])
                for dtb in range(_D // _L):
                    for j in range(_L):
                        dvec = diags[j] + i32(dtb * _L)
                        g0 = plsc.load_gather(r0_v.at[i32(p)], [tokv, dvec])
                        g1 = plsc.load_gather(r1_v.at[i32(p)], [tokv, dvec])
                        plsc.store_scatter(
                            o_v.at[i32(p)],
                            [lax.shift_right_logical(dvec, i32(3)),
                             dvec & i32(7), tokv],
                            w0 * g0 + w1 * g1)
                return koff + i32(_L)

            lax.fori_loop(0, bpw // _L, grp_body, i32(0))

        issue_gathers(i32(0), 0)

        def pair_body(q, r):
            issue_gathers(r + i32(1), 1)
            wait_gathers(r, 0)

            @pl.when(r > i32(0))
            def _():
                pltpu.make_async_copy(*out_args(r - i32(2), 0)).wait()
            combine(r, 0)
            pltpu.async_copy(*out_args(r, 0))
            @pl.when(r + i32(2) < i32(seq))
            def _():
                issue_gathers(r + i32(2), 0)
            wait_gathers(r + i32(1), 1)

            @pl.when(r > i32(0))
            def _():
                pltpu.make_async_copy(*out_args(r - i32(1), 1)).wait()
            combine(r + i32(1), 1)
            pltpu.async_copy(*out_args(r + i32(1), 1))
            return r + i32(2)

        lax.fori_loop(0, seq // 2, pair_body, i32(0))
        pltpu.make_async_copy(*out_args(i32(seq - 2), 0)).wait()
        pltpu.make_async_copy(*out_args(i32(seq - 1), 1)).wait()

    return lookup


def kernel(x, shared_embedding, hash_weights):
    b, s = x.shape
    xt = x.T.astype(jnp.int32)
    wi = hash_weights.reshape(-1, 8, 2).transpose(0, 2, 1).reshape(-1, _L)
    lookup = _make_lookup(s, b)
    out5 = lookup(xt, shared_embedding, wi)
    return jnp.transpose(out5, (2, 4, 0, 1, 3)).reshape(b, s, _D)
